# baseline (speedup 1.0000x reference)
import functools

import jax
import jax.numpy as jnp
from jax import lax
from jax.experimental import pallas as pl
from jax.experimental.pallas import tpu as pltpu
from jax.experimental.pallas import tpu_sc as plsc

NUM_CORES = 2
NUM_SUBCORES = 16
NW = NUM_CORES * NUM_SUBCORES
LANES = 16

BATCH = 16384
EMBED_DIM = 32
B_PER_W = BATCH // NW
CHUNK = 128
N_CHUNKS = B_PER_W // CHUNK


def _sc_kernel(u_idx_hbm, i_idx_hbm, user_hbm, item_hbm, out_hbm,
               uidx_v, iidx_v, urows_v, irows_v, out_v, sem):
    wid = lax.axis_index("s") * NUM_CORES + lax.axis_index("c")
    base = wid * B_PER_W

    pltpu.sync_copy(u_idx_hbm.at[wid], uidx_v)
    pltpu.sync_copy(i_idx_hbm.at[wid], iidx_v)

    copies = []
    for j in range(N_CHUNKS):
        copies.append(pltpu.async_copy(
            user_hbm.at[uidx_v.at[j]], urows_v.at[pl.ds(j * CHUNK, CHUNK)], sem))
        copies.append(pltpu.async_copy(
            item_hbm.at[iidx_v.at[j]], irows_v.at[pl.ds(j * CHUNK, CHUNK)], sem))
    for c in copies:
        c.wait()

    lane = lax.iota(jnp.int32, LANES)

    def group_body(g, _):
        rows = g * LANES + lane
        acc = jnp.zeros((LANES,), jnp.float32)
        for j in range(EMBED_DIM):
            col = jnp.full((LANES,), j, jnp.int32)
            uv = plsc.load_gather(urows_v, [rows, col])
            iv = plsc.load_gather(irows_v, [rows, col])
            acc = acc + uv * iv
        out_v[pl.ds(g * LANES, LANES)] = acc
        return 0

    lax.fori_loop(0, B_PER_W // LANES, group_body, 0)

    pltpu.sync_copy(out_v, out_hbm.at[pl.ds(base, B_PER_W)])


@jax.jit
def _mf_dot(user_indices, item_indices, user_table, item_table):
    mesh = plsc.VectorSubcoreMesh(core_axis_name="c", subcore_axis_name="s")
    kfn = pl.kernel(
        _sc_kernel,
        out_type=jax.ShapeDtypeStruct((BATCH,), jnp.float32),
        mesh=mesh,
        compiler_params=pltpu.CompilerParams(
            needs_layout_passes=False, use_tc_tiling_on_sc=False),
        scratch_types=[
            pltpu.VMEM((N_CHUNKS, CHUNK), jnp.int32),
            pltpu.VMEM((N_CHUNKS, CHUNK), jnp.int32),
            pltpu.VMEM((B_PER_W, EMBED_DIM), jnp.float32),
            pltpu.VMEM((B_PER_W, EMBED_DIM), jnp.float32),
            pltpu.VMEM((B_PER_W,), jnp.float32),
            pltpu.SemaphoreType.DMA,
        ],
    )
    u_idx = user_indices.astype(jnp.int32).reshape(NW, N_CHUNKS, CHUNK)
    i_idx = item_indices.astype(jnp.int32).reshape(NW, N_CHUNKS, CHUNK)
    return kfn(u_idx, i_idx, user_table, item_table)


def kernel(user_indices, item_indices, user_table, item_table):
    return _mf_dot(user_indices, item_indices, user_table, item_table)


# speedup vs baseline: 3.6108x; 3.6108x over previous
import jax
import jax.numpy as jnp
from jax import lax
from jax.experimental import pallas as pl
from jax.experimental.pallas import tpu as pltpu
from jax.experimental.pallas import tpu_sc as plsc

NUM_CORES = 2
NUM_SUBCORES = 16
NW = NUM_CORES * NUM_SUBCORES
LANES = 16

BATCH = 16384
EMBED_DIM = 32
NUM_ROWS = 1000000
B_PER_W = BATCH // NW
KSTAGE = 8
N_STEPS = B_PER_W // KSTAGE


def _sc_kernel(u_idx_hbm, i_idx_hbm, user_t_hbm, item_t_hbm, out_hbm,
               uidx_s, iidx_s, ustage_v, istage_v, out_v, sem):
    wid = lax.axis_index("s") * NUM_CORES + lax.axis_index("c")
    base = wid * B_PER_W

    pltpu.sync_copy(u_idx_hbm.at[wid, 0], uidx_s.at[pl.ds(0, B_PER_W)])
    pltpu.sync_copy(i_idx_hbm.at[wid, 0], iidx_s.at[pl.ds(0, B_PER_W)])

    lane = lax.iota(jnp.int32, LANES)
    d_lo = lax.iota(jnp.int32, LANES)

    def hsum(v):
        for sh in (8, 4, 2, 1):
            v = v + v.at[(lane + sh) & (LANES - 1)].get(
                mode="promise_in_bounds")
        return v

    def step_body(m, carry):
        k0 = m * KSTAGE
        uvec = uidx_s[pl.ds(pl.multiple_of(k0, KSTAGE), LANES)]
        ivec = iidx_s[pl.ds(pl.multiple_of(k0, KSTAGE), LANES)]
        copies = []
        for k in range(KSTAGE):
            u = uvec[k]
            i = ivec[k]
            cu0 = pl.multiple_of((u >> 7) * 128, 128)
            ci0 = pl.multiple_of((i >> 7) * 128, 128)
            copies.append(pltpu.async_copy(
                user_t_hbm.at[:, pl.ds(cu0, 128)], ustage_v.at[k], sem))
            copies.append(pltpu.async_copy(
                item_t_hbm.at[:, pl.ds(ci0, 128)], istage_v.at[k], sem))
        for c in copies:
            c.wait()
        acc = carry
        for k in range(KSTAGE):
            u = uvec[k]
            i = ivec[k]
            cu = jnp.full((LANES,), u & 127, jnp.int32)
            ci = jnp.full((LANES,), i & 127, jnp.int32)
            kk = jnp.full((LANES,), k, jnp.int32)
            uv0 = plsc.load_gather(ustage_v, [kk, d_lo, cu])
            uv1 = plsc.load_gather(ustage_v, [kk, d_lo + LANES, cu])
            iv0 = plsc.load_gather(istage_v, [kk, d_lo, ci])
            iv1 = plsc.load_gather(istage_v, [kk, d_lo + LANES, ci])
            s = hsum(uv0 * iv0 + uv1 * iv1)
            acc = jnp.where(lane == (k0 + k) % LANES, s, acc)
        @pl.when(m % 2 == 1)
        def _():
            out_v[pl.ds(pl.multiple_of((m - 1) * KSTAGE, LANES), LANES)] = acc
        return acc

    lax.fori_loop(0, N_STEPS, step_body, jnp.zeros((LANES,), jnp.float32))

    pltpu.sync_copy(out_v, out_hbm.at[pl.ds(base, B_PER_W)])


@jax.jit
def _mf_dot(user_indices, item_indices, user_table, item_table):
    mesh = plsc.VectorSubcoreMesh(core_axis_name="c", subcore_axis_name="s")
    kfn = pl.kernel(
        _sc_kernel,
        out_type=jax.ShapeDtypeStruct((BATCH,), jnp.float32),
        mesh=mesh,
        compiler_params=pltpu.CompilerParams(
            needs_layout_passes=False, use_tc_tiling_on_sc=True),
        scratch_types=[
            pltpu.VMEM((B_PER_W + LANES,), jnp.int32),
            pltpu.VMEM((B_PER_W + LANES,), jnp.int32),
            pltpu.VMEM((KSTAGE, EMBED_DIM, 128), jnp.float32),
            pltpu.VMEM((KSTAGE, EMBED_DIM, 128), jnp.float32),
            pltpu.VMEM((B_PER_W,), jnp.float32),
            pltpu.SemaphoreType.DMA,
        ],
    )
    u_idx = user_indices.astype(jnp.int32).reshape(NW, 1, B_PER_W)
    i_idx = item_indices.astype(jnp.int32).reshape(NW, 1, B_PER_W)
    return kfn(u_idx, i_idx, user_table.T, item_table.T)


def kernel(user_indices, item_indices, user_table, item_table):
    return _mf_dot(user_indices, item_indices, user_table, item_table)
